# baseline (speedup 1.0000x reference)
import functools

import jax
import jax.numpy as jnp
from jax import lax
from jax.experimental import pallas as pl
from jax.experimental.pallas import tpu as pltpu
from jax.experimental.pallas import tpu_sc as plsc

_NC = 2
_NS = 16
_NW = _NC * _NS
_NSLOT = 4


def _make_gather_kernel(B, T, D):
    TPW = T // _NW
    mesh = plsc.VectorSubcoreMesh(core_axis_name="c", subcore_axis_name="s")

    @functools.partial(
        pl.kernel,
        out_type=jax.ShapeDtypeStruct((B * T, D), jnp.float32),
        mesh=mesh,
        scratch_types=[
            pltpu.VMEM((TPW,), jnp.int32),
            pltpu.VMEM((B, TPW), jnp.int32),
            pltpu.VMEM((_NSLOT, TPW, D), jnp.float32),
            pltpu.SemaphoreType.DMA,
            pltpu.SemaphoreType.DMA,
        ],
    )
    def gather_kernel(img_hbm, inv_hbm, out_hbm,
                      rawinv_v, flatidx_v, buf_v, sem_in, sem_out):
        c = lax.axis_index("c")
        s = lax.axis_index("s")
        wid = s * _NC + c
        base = wid * TPW

        pltpu.sync_copy(inv_hbm.at[pl.ds(base, TPW)], rawinv_v)

        def fill(b, carry):
            for t0 in range(0, TPW, 16):
                flatidx_v[b, pl.ds(t0, 16)] = rawinv_v[pl.ds(t0, 16)] + b * T
            return carry
        lax.fori_loop(0, B, fill, 0)

        def in_copy(b, slot):
            return pltpu.make_async_copy(
                img_hbm.at[flatidx_v.at[b]], buf_v.at[slot], sem_in)

        def out_copy(b, slot):
            return pltpu.make_async_copy(
                buf_v.at[slot], out_hbm.at[pl.ds(b * T + base, TPW)], sem_out)

        for j in range(_NSLOT):
            in_copy(j, j).start()

        def step(g, carry):
            for j in range(_NSLOT):
                b = g * _NSLOT + j
                in_copy(b, j).wait()
                out_copy(b, j).start()
                out_copy(b, j).wait()
                in_copy(b + _NSLOT, j).start()
            return carry
        lax.fori_loop(0, B // _NSLOT - 1, step, 0)

        blast = B - _NSLOT
        for j in range(_NSLOT):
            in_copy(blast + j, j).wait()
            out_copy(blast + j, j).start()
        for j in range(_NSLOT):
            out_copy(blast + j, j).wait()

    return gather_kernel


def kernel(img, index_flat_inv):
    B, T, D = img.shape
    img_flat = img.reshape(B * T, D)
    idx = index_flat_inv.astype(jnp.int32)
    inv = jnp.zeros((T,), jnp.int32).at[idx].set(
        jnp.arange(T, dtype=jnp.int32))
    out_flat = _make_gather_kernel(B, T, D)(img_flat, inv)
    return out_flat.reshape(B, T, D)


# speedup vs baseline: 1.0354x; 1.0354x over previous
import functools

import jax
import jax.numpy as jnp
from jax import lax
from jax.experimental import pallas as pl
from jax.experimental.pallas import tpu as pltpu
from jax.experimental.pallas import tpu_sc as plsc

_NC = 2
_NS = 16
_NW = _NC * _NS
_NSLOT = 4


def _make_scatter_kernel(B, T, D):
    TPW = T // _NW
    mesh = plsc.VectorSubcoreMesh(core_axis_name="c", subcore_axis_name="s")

    @functools.partial(
        pl.kernel,
        out_type=jax.ShapeDtypeStruct((B * T, D), jnp.float32),
        mesh=mesh,
        scratch_types=[
            pltpu.VMEM((TPW,), jnp.int32),
            pltpu.VMEM((B, TPW), jnp.int32),
            pltpu.VMEM((_NSLOT, TPW, D), jnp.float32),
            pltpu.SemaphoreType.DMA,
            pltpu.SemaphoreType.DMA,
        ],
    )
    def scatter_kernel(img_hbm, idx_hbm, out_hbm,
                       rawidx_v, flatidx_v, buf_v, sem_in, sem_out):
        c = lax.axis_index("c")
        s = lax.axis_index("s")
        wid = s * _NC + c
        base = wid * TPW

        def in_copy(b, slot):
            return pltpu.make_async_copy(
                img_hbm.at[pl.ds(b * T + base, TPW)], buf_v.at[slot], sem_in)

        def out_copy(b, slot):
            return pltpu.make_async_copy(
                buf_v.at[slot], out_hbm.at[flatidx_v.at[b]], sem_out)

        for j in range(_NSLOT):
            in_copy(j, j).start()

        pltpu.sync_copy(idx_hbm.at[pl.ds(base, TPW)], rawidx_v)

        def fill(b, carry):
            for t0 in range(0, TPW, 16):
                flatidx_v[b, pl.ds(t0, 16)] = rawidx_v[pl.ds(t0, 16)] + b * T
            return carry
        lax.fori_loop(0, B, fill, 0)

        def step(g, carry):
            for j in range(_NSLOT):
                b = g * _NSLOT + j
                in_copy(b, j).wait()
                out_copy(b, j).start()
                out_copy(b, j).wait()
                in_copy(b + _NSLOT, j).start()
            return carry
        lax.fori_loop(0, B // _NSLOT - 1, step, 0)

        blast = B - _NSLOT
        for j in range(_NSLOT):
            in_copy(blast + j, j).wait()
            out_copy(blast + j, j).start()
        for j in range(_NSLOT):
            out_copy(blast + j, j).wait()

    return scatter_kernel


def kernel(img, index_flat_inv):
    B, T, D = img.shape
    img_flat = img.reshape(B * T, D)
    idx = index_flat_inv.astype(jnp.int32)
    out_flat = _make_scatter_kernel(B, T, D)(img_flat, idx)
    return out_flat.reshape(B, T, D)
